# baseline (speedup 1.0000x reference)
import functools

import jax
import jax.numpy as jnp
from jax import lax
from jax.experimental import pallas as pl
from jax.experimental.pallas import tpu as pltpu
from jax.experimental.pallas import tpu_sc as plsc

CHUNK = 128
NW = 32
SPLITS = 4
BLK = 2560


def _gather_rows(idxa, idxb, table):
    Ms = idxa.shape[0]
    V, D = table.shape
    tw = Ms // NW
    mc = tw // CHUNK

    mesh = plsc.VectorSubcoreMesh(core_axis_name="c", subcore_axis_name="s")

    @functools.partial(
        pl.kernel,
        mesh=mesh,
        compiler_params=pltpu.CompilerParams(use_tc_tiling_on_sc=False),
        out_type=jax.ShapeDtypeStruct((Ms, 2 * D), jnp.float32),
        scratch_types=[
            pltpu.VMEM((2 * tw,), jnp.int32),
            pltpu.VMEM((CHUNK, D), jnp.float32),
            pltpu.VMEM((CHUNK, D), jnp.float32),
            pltpu.SemaphoreType.DMA,
            pltpu.SemaphoreType.DMA,
        ],
    )
    def gather_kernel(idxa_hbm, idxb_hbm, table_hbm, e_hbm, idx_v,
                      rows_a, rows_b, sem_a, sem_b):
        wid = lax.axis_index("s") * 2 + lax.axis_index("c")
        tok_base = wid * tw
        pltpu.sync_copy(idxa_hbm.at[pl.ds(tok_base, tw)],
                        idx_v.at[pl.ds(0, tw)])
        pltpu.sync_copy(idxb_hbm.at[pl.ds(tok_base, tw)],
                        idx_v.at[pl.ds(tw, tw)])

        def body(j, carry):
            cp_a = pltpu.async_copy(
                table_hbm.at[idx_v.at[pl.ds(j * CHUNK, CHUNK)]], rows_a, sem_a)
            cp_b = pltpu.async_copy(
                table_hbm.at[idx_v.at[pl.ds(tw + j * CHUNK, CHUNK)]], rows_b,
                sem_b)
            cp_a.wait()
            cp_b.wait()
            row0 = tok_base + j * CHUNK
            pltpu.sync_copy(rows_a, e_hbm.at[pl.ds(row0, CHUNK), pl.ds(0, D)])
            pltpu.sync_copy(rows_b, e_hbm.at[pl.ds(row0, CHUNK), pl.ds(D, D)])
            return carry

        lax.fori_loop(0, mc, body, 0)

    return gather_kernel(idxa, idxb, table)


def _mm_first_body(h, e_ref, w_ref, o_ref):
    d = lax.dot_general(
        e_ref[...], w_ref[...],
        dimension_numbers=(((1,), (1,)), ((), ())),
        preferred_element_type=jnp.float32,
    )
    o_ref[0] = d[:, :h]
    o_ref[1] = d[:, h:]


def _mm_next_body(h, prev_ref, e_ref, w_ref, o_ref):
    del prev_ref
    _mm_first_body(h, e_ref, w_ref, o_ref)


def _project_slice(e2s, W2, prev, s, M):
    Ms, K = e2s.shape
    H = W2.shape[0] // 2
    steps = Ms // BLK
    row_off = s * steps
    common = dict(
        grid=(steps,),
        out_specs=pl.BlockSpec((2, BLK, H), lambda i: (0, i + row_off, 0)),
        out_shape=jax.ShapeDtypeStruct((2, M, H), jnp.float32),
    )
    e_spec = pl.BlockSpec((BLK, K), lambda i: (i, 0))
    w_spec = pl.BlockSpec((2 * H, K), lambda i: (0, 0))
    if prev is None:
        return pl.pallas_call(
            functools.partial(_mm_first_body, H),
            in_specs=[e_spec, w_spec],
            **common,
        )(e2s, W2)
    return pl.pallas_call(
        functools.partial(_mm_next_body, H),
        in_specs=[pl.BlockSpec(memory_space=pltpu.MemorySpace.HBM),
                  e_spec, w_spec],
        input_output_aliases={0: 0},
        **common,
    )(prev, e2s, W2)


def kernel(x, table, W):
    B, L = x.shape
    N = B * L
    H, D = W.shape
    M = N // 2
    W2 = jnp.zeros((2 * H, 2 * D), W.dtype)
    W2 = W2.at[:H, :D].set(W).at[H:, D:].set(W)
    rows_s = B // (2 * SPLITS)
    xi = x.astype(jnp.int32)
    out = None
    z = (W[0, 0] * 0.0).astype(jnp.int32)
    for s in range(SPLITS):
        idxa = xi[s * rows_s:(s + 1) * rows_s].reshape(-1) + z
        idxb = xi[B // 2 + s * rows_s:B // 2 + (s + 1) * rows_s].reshape(-1) + z
        e2s = _gather_rows(idxa, idxb, table)
        out = _project_slice(e2s, W2, out, s, M)
        z = (e2s[0, 0] * 0.0).astype(jnp.int32)
    return out.reshape(B, L, H)


# speedup vs baseline: 1.0249x; 1.0249x over previous
import functools

import jax
import jax.numpy as jnp
from jax import lax
from jax.experimental import pallas as pl
from jax.experimental.pallas import tpu as pltpu
from jax.experimental.pallas import tpu_sc as plsc

CHUNK = 128
NW = 32
SPLITS = 4
BLK = 2560


def _gather_rows(idxa, idxb, table):
    Ms = idxa.shape[0]
    V, D = table.shape
    tw = Ms // NW
    mc = tw // CHUNK

    mesh = plsc.VectorSubcoreMesh(core_axis_name="c", subcore_axis_name="s")

    @functools.partial(
        pl.kernel,
        mesh=mesh,
        compiler_params=pltpu.CompilerParams(use_tc_tiling_on_sc=False),
        out_type=jax.ShapeDtypeStruct((Ms, 2 * D), jnp.float32),
        scratch_types=[
            pltpu.VMEM((2 * tw,), jnp.int32),
            pltpu.VMEM((CHUNK, D), jnp.float32),
            pltpu.VMEM((CHUNK, D), jnp.float32),
            pltpu.SemaphoreType.DMA,
            pltpu.SemaphoreType.DMA,
        ],
    )
    def gather_kernel(idxa_hbm, idxb_hbm, table_hbm, e_hbm, idx_v,
                      rows_a, rows_b, sem_a, sem_b):
        wid = lax.axis_index("s") * 2 + lax.axis_index("c")
        tok_base = wid * tw
        pltpu.sync_copy(idxa_hbm.at[pl.ds(tok_base, tw)],
                        idx_v.at[pl.ds(0, tw)])
        pltpu.sync_copy(idxb_hbm.at[pl.ds(tok_base, tw)],
                        idx_v.at[pl.ds(tw, tw)])

        def body(j, carry):
            cp_a = pltpu.async_copy(
                table_hbm.at[idx_v.at[pl.ds(j * CHUNK, CHUNK)]], rows_a, sem_a)
            cp_b = pltpu.async_copy(
                table_hbm.at[idx_v.at[pl.ds(tw + j * CHUNK, CHUNK)]], rows_b,
                sem_b)
            cp_a.wait()
            cp_b.wait()
            row0 = tok_base + j * CHUNK
            pltpu.sync_copy(rows_a, e_hbm.at[pl.ds(row0, CHUNK), pl.ds(0, D)])
            pltpu.sync_copy(rows_b, e_hbm.at[pl.ds(row0, CHUNK), pl.ds(D, D)])
            return carry

        lax.fori_loop(0, mc, body, 0)

    return gather_kernel(idxa, idxb, table)


def _mm_first_body(h, e_ref, w_ref, o_ref):
    d = lax.dot_general(
        e_ref[...], w_ref[...],
        dimension_numbers=(((1,), (1,)), ((), ())),
        preferred_element_type=jnp.float32,
    )
    o_ref[0] = d[:, :h]
    o_ref[1] = d[:, h:]


def _mm_next_body(h, prev_ref, e_ref, w_ref, o_ref):
    del prev_ref
    _mm_first_body(h, e_ref, w_ref, o_ref)


def _project_slice(e2s, W2, prev, s, M):
    Ms, K = e2s.shape
    H = W2.shape[0] // 2
    steps = Ms // BLK
    row_off = s * steps
    common = dict(
        grid=(steps,),
        out_specs=pl.BlockSpec((2, BLK, H), lambda i: (0, i + row_off, 0)),
        out_shape=jax.ShapeDtypeStruct((2, M, H), jnp.float32),
    )
    e_spec = pl.BlockSpec((BLK, K), lambda i: (i, 0))
    w_spec = pl.BlockSpec((2 * H, K), lambda i: (0, 0))
    if prev is None:
        return pl.pallas_call(
            functools.partial(_mm_first_body, H),
            in_specs=[e_spec, w_spec],
            **common,
        )(e2s, W2)
    return pl.pallas_call(
        functools.partial(_mm_next_body, H),
        in_specs=[pl.BlockSpec(memory_space=pltpu.MemorySpace.HBM),
                  e_spec, w_spec],
        input_output_aliases={0: 0},
        **common,
    )(prev, e2s, W2)


def kernel(x, table, W):
    B, L = x.shape
    N = B * L
    H, D = W.shape
    M = N // 2
    W2 = jnp.zeros((2 * H, 2 * D), W.dtype)
    W2 = W2.at[:H, :D].set(W).at[H:, D:].set(W)
    rows_s = B // (2 * SPLITS)
    xi = x.astype(jnp.int32)
    out = None
    z = (W[0, 0] * 0.0).astype(jnp.int32)
    for s in range(SPLITS):
        idxa = xi[s * rows_s:(s + 1) * rows_s].reshape(-1) + z
        idxb = xi[B // 2 + s * rows_s:B // 2 + (s + 1) * rows_s].reshape(-1) + z
        e2s = _gather_rows(idxa, idxb, table)
        out = _project_slice(e2s, W2, out, s, M)
    return out.reshape(B, L, H)


# speedup vs baseline: 1.0269x; 1.0020x over previous
import functools

import jax
import jax.numpy as jnp
from jax import lax
from jax.experimental import pallas as pl
from jax.experimental.pallas import tpu as pltpu
from jax.experimental.pallas import tpu_sc as plsc

CHUNK = 128
NW = 32
SPLITS = 4
BLK = 3200


def _gather_rows(idxa, idxb, table):
    Ms = idxa.shape[0]
    V, D = table.shape
    tw = Ms // NW
    mc = tw // CHUNK

    mesh = plsc.VectorSubcoreMesh(core_axis_name="c", subcore_axis_name="s")

    @functools.partial(
        pl.kernel,
        mesh=mesh,
        compiler_params=pltpu.CompilerParams(use_tc_tiling_on_sc=False),
        out_type=jax.ShapeDtypeStruct((Ms, 2 * D), jnp.float32),
        scratch_types=[
            pltpu.VMEM((2 * tw,), jnp.int32),
            pltpu.VMEM((CHUNK, D), jnp.float32),
            pltpu.VMEM((CHUNK, D), jnp.float32),
            pltpu.SemaphoreType.DMA,
            pltpu.SemaphoreType.DMA,
        ],
    )
    def gather_kernel(idxa_hbm, idxb_hbm, table_hbm, e_hbm, idx_v,
                      rows_a, rows_b, sem_a, sem_b):
        wid = lax.axis_index("s") * 2 + lax.axis_index("c")
        tok_base = wid * tw
        pltpu.sync_copy(idxa_hbm.at[pl.ds(tok_base, tw)],
                        idx_v.at[pl.ds(0, tw)])
        pltpu.sync_copy(idxb_hbm.at[pl.ds(tok_base, tw)],
                        idx_v.at[pl.ds(tw, tw)])

        def body(j, carry):
            cp_a = pltpu.async_copy(
                table_hbm.at[idx_v.at[pl.ds(j * CHUNK, CHUNK)]], rows_a, sem_a)
            cp_b = pltpu.async_copy(
                table_hbm.at[idx_v.at[pl.ds(tw + j * CHUNK, CHUNK)]], rows_b,
                sem_b)
            cp_a.wait()
            cp_b.wait()
            row0 = tok_base + j * CHUNK
            pltpu.sync_copy(rows_a, e_hbm.at[pl.ds(row0, CHUNK), pl.ds(0, D)])
            pltpu.sync_copy(rows_b, e_hbm.at[pl.ds(row0, CHUNK), pl.ds(D, D)])
            return carry

        lax.fori_loop(0, mc, body, 0)

    return gather_kernel(idxa, idxb, table)


def _mm_first_body(h, e_ref, w_ref, o_ref):
    d = lax.dot_general(
        e_ref[...], w_ref[...],
        dimension_numbers=(((1,), (1,)), ((), ())),
        preferred_element_type=jnp.float32,
    )
    o_ref[0] = d[:, :h]
    o_ref[1] = d[:, h:]


def _mm_next_body(h, prev_ref, e_ref, w_ref, o_ref):
    del prev_ref
    _mm_first_body(h, e_ref, w_ref, o_ref)


def _project_slice(e2s, W2, prev, s, M):
    Ms, K = e2s.shape
    H = W2.shape[0] // 2
    steps = Ms // BLK
    row_off = s * steps
    common = dict(
        grid=(steps,),
        out_specs=pl.BlockSpec((2, BLK, H), lambda i: (0, i + row_off, 0)),
        out_shape=jax.ShapeDtypeStruct((2, M, H), jnp.float32),
    )
    e_spec = pl.BlockSpec((BLK, K), lambda i: (i, 0))
    w_spec = pl.BlockSpec((2 * H, K), lambda i: (0, 0))
    if prev is None:
        return pl.pallas_call(
            functools.partial(_mm_first_body, H),
            in_specs=[e_spec, w_spec],
            **common,
        )(e2s, W2)
    return pl.pallas_call(
        functools.partial(_mm_next_body, H),
        in_specs=[pl.BlockSpec(memory_space=pltpu.MemorySpace.HBM),
                  e_spec, w_spec],
        input_output_aliases={0: 0},
        **common,
    )(prev, e2s, W2)


def kernel(x, table, W):
    B, L = x.shape
    N = B * L
    H, D = W.shape
    M = N // 2
    W2 = jnp.zeros((2 * H, 2 * D), W.dtype)
    W2 = W2.at[:H, :D].set(W).at[H:, D:].set(W)
    rows_s = B // (2 * SPLITS)
    xi = x.astype(jnp.int32)
    out = None
    z = (W[0, 0] * 0.0).astype(jnp.int32)
    for s in range(SPLITS):
        idxa = xi[s * rows_s:(s + 1) * rows_s].reshape(-1) + z
        idxb = xi[B // 2 + s * rows_s:B // 2 + (s + 1) * rows_s].reshape(-1) + z
        e2s = _gather_rows(idxa, idxb, table)
        out = _project_slice(e2s, W2, out, s, M)
    return out.reshape(B, L, H)
